# baseline (speedup 1.0000x reference)
import functools

import jax
import jax.numpy as jnp
from jax import lax
from jax.experimental import pallas as pl
from jax.experimental.pallas import tpu as pltpu
from jax.experimental.pallas import tpu_sc as plsc

_NUM_VOCAB = 1000000
_MAXLEN = 200
_NUM_HID = 64
_BATCH = 4096
_SEQ = 200

_NC = 2
_NS = 16
_NW = _NC * _NS
_TOTAL = _BATCH * _SEQ
_ROWS_PER_W = _TOTAL // _NW
_CHUNK = 800
_NCHUNK = _ROWS_PER_W // _CHUNK
_REPS = _CHUNK // _MAXLEN
_LANES = 16
_SLICES = _NUM_HID // _LANES


def _body(x_hbm, emb_hbm, pos_hbm, out_hbm,
          pos_v, idx0, idx1, tok0, tok1, g0, g1, o0, o1):
    cid = lax.axis_index("c")
    sid = lax.axis_index("s")
    wid = sid * _NC + cid
    base = pl.multiple_of(wid * _ROWS_PER_W, _CHUNK)

    pltpu.sync_copy(pos_hbm, pos_v)

    bufs = ((idx0, tok0, g0, o0), (idx1, tok1, g1, o1))

    def gstart(g, idx_v, tok_v, gsem):
        off = pl.multiple_of(base + g * _CHUNK, _CHUNK)
        pltpu.sync_copy(x_hbm.at[pl.ds(off, _CHUNK)], idx_v)
        pltpu.async_copy(emb_hbm.at[idx_v], tok_v, gsem)

    def gwait(idx_v, tok_v, gsem):
        pltpu.make_async_copy(emb_hbm.at[idx_v], tok_v, gsem).wait()

    def add_pos(tok_v):
        def srow(s, carry):
            for c in range(_SLICES):
                pv = pos_v[s, pl.ds(c * _LANES, _LANES)]
                for rep in range(_REPS):
                    plsc.addupdate(
                        tok_v.at[rep * _MAXLEN + s, pl.ds(c * _LANES, _LANES)],
                        pv)
            return carry
        lax.fori_loop(0, _MAXLEN, srow, 0)

    gstart(0, idx0, tok0, g0)
    gstart(1, idx1, tok1, g1)

    def step(i, carry):
        for b, (idx_v, tok_v, gsem, osem) in enumerate(bufs):
            g = 2 * i + b
            off = pl.multiple_of(base + g * _CHUNK, _CHUNK)
            gwait(idx_v, tok_v, gsem)
            add_pos(tok_v)
            pltpu.async_copy(tok_v, out_hbm.at[pl.ds(off, _CHUNK)], osem)

            nxt = g + 2

            @pl.when(nxt < _NCHUNK)
            def _():
                pltpu.make_async_copy(
                    tok_v, out_hbm.at[pl.ds(off, _CHUNK)], osem).wait()
                gstart(nxt, idx_v, tok_v, gsem)
        return carry

    lax.fori_loop(0, _NCHUNK // 2, step, 0)

    for idx_v, tok_v, gsem, osem in bufs:
        pltpu.make_async_copy(
            tok_v, out_hbm.at[pl.ds(base, _CHUNK)], osem).wait()


_mesh = plsc.VectorSubcoreMesh(core_axis_name="c", subcore_axis_name="s")

_tok_kernel = functools.partial(
    pl.kernel,
    mesh=_mesh,
    compiler_params=pltpu.CompilerParams(use_tc_tiling_on_sc=False),
    out_type=jax.ShapeDtypeStruct((_TOTAL, _NUM_HID), jnp.float32),
    scratch_types=[
        pltpu.VMEM((_MAXLEN, _NUM_HID), jnp.float32),
        pltpu.VMEM((_CHUNK,), jnp.int32),
        pltpu.VMEM((_CHUNK,), jnp.int32),
        pltpu.VMEM((_CHUNK, _NUM_HID), jnp.float32),
        pltpu.VMEM((_CHUNK, _NUM_HID), jnp.float32),
        pltpu.SemaphoreType.DMA,
        pltpu.SemaphoreType.DMA,
        pltpu.SemaphoreType.DMA,
        pltpu.SemaphoreType.DMA,
    ],
)(_body)


@jax.jit
def kernel(x, emb_table, pos_table):
    out = _tok_kernel(x.reshape(-1).astype(jnp.int32), emb_table, pos_table)
    return out.reshape(_BATCH, _SEQ, _NUM_HID)
